# baseline (speedup 1.0000x reference)
import functools

import jax
import jax.numpy as jnp
from jax import lax
from jax.experimental import pallas as pl
from jax.experimental.pallas import tpu as pltpu
from jax.experimental.pallas import tpu_sc as plsc

NDIM = 768
NCLASS = 16
NBIN = 100
NTOK = 4096
NBIN_PAD = 128
NROWS = NCLASS * NDIM

NWORKER = 32
DPW = NDIM // NWORKER
TCHUNK = 64
NCHUNK = NTOK // TCHUNK

_SORT_ROWS = 512
_MM_T = 512


_SROWS = NCLASS * DPW


def _sort_exp_body(kx_ref, ky_ref, ld_ref, sx_ref, sy_ref, dd_ref):
    ids = lax.broadcasted_iota(jnp.int32, (_SROWS, NBIN_PAD), 1)
    fill = jnp.full((_SROWS, NBIN_PAD - NBIN), jnp.inf, jnp.float32)

    def bitonic(x):
        x = jnp.concatenate([x.reshape(_SROWS, NBIN), fill], axis=1)
        for k in (2, 4, 8, 16, 32, 64, 128):
            j = k // 2
            while j >= 1:
                pj = pltpu.roll(x, NBIN_PAD - j, axis=1)
                mj = pltpu.roll(x, j, axis=1)
                low = (ids & j) == 0
                partner = jnp.where(low, pj, mj)
                keep_min = low == ((ids & k) == 0)
                x = jnp.where(keep_min, jnp.minimum(x, partner),
                              jnp.maximum(x, partner))
                j //= 2
        return x[:, :NBIN].reshape(1, NCLASS, DPW, NBIN)

    sx_ref[...] = bitonic(kx_ref[...])
    sy_ref[...] = bitonic(ky_ref[...])
    dd_ref[...] = jnp.exp(ld_ref[...]).reshape(1, NCLASS, DPW, NBIN)


def _mm_body(w_ref, a_ref, o_ref):
    o_ref[...] = lax.dot_general(
        w_ref[...], a_ref[...], (((0,), (1,)), ((), ())),
        preferred_element_type=jnp.float32)


def _final_body(data_ref, yt_ref, d0t_ref, dvt_ref, w_ref, o_ref, lj_ref):
    diff = yt_ref[...] - d0t_ref[...]
    o_ref[...] = data_ref[...] + lax.dot_general(
        diff, w_ref[...], (((0,), (1,)), ((), ())),
        preferred_element_type=jnp.float32)
    lj_ref[...] = jnp.sum(jnp.log(dvt_ref[...]), axis=0)[None, None, :]


def _spline_sc_body(d0_hbm, par_hbm, sx_hbm, sy_hbm, dd_hbm, y_hbm, dv_hbm,
                    sx_v, sy_v, dd_v, par_v, x_v, yo_v, dv_v):
    wid = lax.axis_index("s") * 2 + lax.axis_index("c")
    dbase = wid * DPW
    pltpu.sync_copy(par_hbm, par_v)
    pltpu.sync_copy(sx_hbm.at[wid], sx_v.at[pl.ds(0, NCLASS * DPW * NBIN)])
    pltpu.sync_copy(sy_hbm.at[wid], sy_v)
    pltpu.sync_copy(dd_hbm.at[wid], dd_v)
    ngrp = (TCHUNK // 16) * DPW

    def chunk_body(ci, carry):
        t0 = ci * TCHUNK
        pltpu.sync_copy(d0_hbm.at[pl.ds(dbase, DPW), pl.ds(t0, TCHUNK)], x_v)

        @plsc.parallel_loop(0, ngrp, unroll=8)
        def grp_body(g):
            tv = lax.rem(g, TCHUNK // 16)
            dl = lax.div(g, TCHUNK // 16)
            toff = tv * 16
            c_vec = par_v[pl.ds(t0 + toff, 16)]
            base = c_vec * (DPW * NBIN) + dl * NBIN
            v = x_v[dl, pl.ds(toff, 16)]
            idxp = jnp.zeros((16,), jnp.int32)
            basem1 = base - 1
            for b in (64, 32, 16, 8, 4, 2, 1):
                jj = idxp + b
                xprobe = plsc.load_gather(sx_v, [basem1 + jj])
                take = xprobe < v
                if b <= 16:
                    take &= jj <= NBIN
                idxp = jnp.where(take, jj, idxp)
            kk = jnp.clip(idxp - 1, 0, NBIN - 2)
            bk = base + kk
            bk1 = bk + 1
            xk = plsc.load_gather(sx_v, [bk])
            xk1 = plsc.load_gather(sx_v, [bk1])
            yk = plsc.load_gather(sy_v, [bk])
            yk1 = plsc.load_gather(sy_v, [bk1])
            dk = plsc.load_gather(dd_v, [bk])
            dk1 = plsc.load_gather(dd_v, [bk1])
            w = xk1 - xk
            s = (yk1 - yk) / w
            xi = jnp.clip((v - xk) / w, 0.0, 1.0)
            omxi = 1.0 - xi
            xio = xi * omxi
            denom = s + (dk1 + dk - 2.0 * s) * xio
            y_sp = yk + (yk1 - yk) * (s * xi * xi + dk * xio) / denom
            deriv_sp = (s * s
                        * (dk1 * xi * xi + 2.0 * s * xio + dk * omxi * omxi)
                        / (denom * denom))
            below = (idxp == 0) & (v < xk)
            above = idxp >= NBIN
            y_out = jnp.where(below, yk + (v - xk) * dk,
                              jnp.where(above, yk1 + (v - xk1) * dk1, y_sp))
            d_out = jnp.where(below, dk, jnp.where(above, dk1, deriv_sp))
            yo_v[dl, pl.ds(toff, 16)] = y_out
            dv_v[dl, pl.ds(toff, 16)] = d_out

        pltpu.sync_copy(yo_v, y_hbm.at[pl.ds(dbase, DPW), pl.ds(t0, TCHUNK)])
        pltpu.sync_copy(dv_v, dv_hbm.at[pl.ds(dbase, DPW), pl.ds(t0, TCHUNK)])
        return carry

    lax.fori_loop(0, NCHUNK, chunk_body, 0)


def kernel(data, param, wT, knots_x, knots_y, log_deriv):
    param32 = param.astype(jnp.int32)
    kx4 = knots_x.reshape(NCLASS, NWORKER, DPW, NBIN)
    ky4 = knots_y.reshape(NCLASS, NWORKER, DPW, NBIN)
    ld4 = log_deriv.reshape(NCLASS, NWORKER, DPW, NBIN)

    iblk = pl.BlockSpec((NCLASS, 1, DPW, NBIN), lambda i: (0, i, 0, 0))
    oblk = pl.BlockSpec((1, NCLASS, DPW, NBIN), lambda i: (i, 0, 0, 0))
    sx, sy, dd = pl.pallas_call(
        _sort_exp_body,
        grid=(NWORKER,),
        in_specs=[iblk, iblk, iblk],
        out_specs=[oblk, oblk, oblk],
        out_shape=[jax.ShapeDtypeStruct((NWORKER, NCLASS, DPW, NBIN),
                                        jnp.float32)] * 3,
    )(kx4, ky4, ld4)
    sx = sx.reshape(NWORKER, NCLASS * DPW * NBIN)
    sy = sy.reshape(NWORKER, NCLASS * DPW * NBIN)
    dd = dd.reshape(NWORKER, NCLASS * DPW * NBIN)

    data0t = pl.pallas_call(
        _mm_body,
        grid=(NTOK // _MM_T,),
        in_specs=[pl.BlockSpec((NDIM, NDIM), lambda i: (0, 0)),
                  pl.BlockSpec((_MM_T, NDIM), lambda i: (i, 0))],
        out_specs=pl.BlockSpec((NDIM, _MM_T), lambda i: (0, i)),
        out_shape=jax.ShapeDtypeStruct((NDIM, NTOK), jnp.float32),
    )(wT, data)

    spline = pl.kernel(
        _spline_sc_body,
        out_type=[jax.ShapeDtypeStruct((NDIM, NTOK), jnp.float32),
                  jax.ShapeDtypeStruct((NDIM, NTOK), jnp.float32)],
        mesh=plsc.VectorSubcoreMesh(core_axis_name="c", subcore_axis_name="s"),
        compiler_params=pltpu.CompilerParams(use_tc_tiling_on_sc=False,
                                             needs_layout_passes=False),
        scratch_types=[
            pltpu.VMEM((NCLASS * DPW * NBIN + 32,), jnp.float32),
            pltpu.VMEM((NCLASS * DPW * NBIN,), jnp.float32),
            pltpu.VMEM((NCLASS * DPW * NBIN,), jnp.float32),
            pltpu.VMEM((NTOK,), jnp.int32),
            pltpu.VMEM((DPW, TCHUNK), jnp.float32),
            pltpu.VMEM((DPW, TCHUNK), jnp.float32),
            pltpu.VMEM((DPW, TCHUNK), jnp.float32),
        ],
    )
    yt, dvt = spline(data0t, param32, sx, sy, dd)

    data_out, lj = pl.pallas_call(
        _final_body,
        grid=(NTOK // _MM_T,),
        in_specs=[pl.BlockSpec((_MM_T, NDIM), lambda i: (i, 0)),
                  pl.BlockSpec((NDIM, _MM_T), lambda i: (0, i)),
                  pl.BlockSpec((NDIM, _MM_T), lambda i: (0, i)),
                  pl.BlockSpec((NDIM, _MM_T), lambda i: (0, i)),
                  pl.BlockSpec((NDIM, NDIM), lambda i: (0, 0))],
        out_specs=[pl.BlockSpec((_MM_T, NDIM), lambda i: (i, 0)),
                   pl.BlockSpec((1, 1, _MM_T), lambda i: (i, 0, 0))],
        out_shape=[jax.ShapeDtypeStruct((NTOK, NDIM), jnp.float32),
                   jax.ShapeDtypeStruct((NTOK // _MM_T, 1, _MM_T), jnp.float32)],
    )(data, yt, data0t, dvt, wT)
    return data_out, lj.reshape(NTOK)


# speedup vs baseline: 1.1201x; 1.1201x over previous
import functools

import jax
import jax.numpy as jnp
from jax import lax
from jax.experimental import pallas as pl
from jax.experimental.pallas import tpu as pltpu
from jax.experimental.pallas import tpu_sc as plsc

NDIM = 768
NCLASS = 16
NBIN = 100
NTOK = 4096
NBIN_PAD = 128
NROWS = NCLASS * NDIM

NWORKER = 32
DPW = NDIM // NWORKER
TCHUNK = 64
NCHUNK = NTOK // TCHUNK

_SORT_ROWS = 512
_MM_T = 512


_SROWS = NCLASS * DPW


def _sort_exp_body(kx_ref, ky_ref, ld_ref, sx_ref, sy_ref, dd_ref):
    ids = lax.broadcasted_iota(jnp.int32, (_SROWS, NBIN_PAD), 1)
    fill = jnp.full((_SROWS, NBIN_PAD - NBIN), jnp.inf, jnp.float32)

    def bitonic(x):
        x = jnp.concatenate([x.reshape(_SROWS, NBIN), fill], axis=1)
        for k in (2, 4, 8, 16, 32, 64, 128):
            j = k // 2
            while j >= 1:
                pj = pltpu.roll(x, NBIN_PAD - j, axis=1)
                mj = pltpu.roll(x, j, axis=1)
                low = (ids & j) == 0
                partner = jnp.where(low, pj, mj)
                keep_min = low == ((ids & k) == 0)
                x = jnp.where(keep_min, jnp.minimum(x, partner),
                              jnp.maximum(x, partner))
                j //= 2
        return x[:, :NBIN].reshape(1, NCLASS, DPW, NBIN)

    sx_ref[...] = bitonic(kx_ref[...])
    sy_ref[...] = bitonic(ky_ref[...])
    dd_ref[...] = jnp.exp(ld_ref[...]).reshape(1, NCLASS, DPW, NBIN)


def _mm_body(w_ref, a_ref, o_ref):
    o_ref[...] = lax.dot_general(
        w_ref[...], a_ref[...], (((0,), (1,)), ((), ())),
        preferred_element_type=jnp.float32)


def _final_body(data_ref, yt_ref, d0t_ref, dvt_ref, w_ref, o_ref, lj_ref):
    diff = yt_ref[...] - d0t_ref[...]
    o_ref[...] = data_ref[...] + lax.dot_general(
        diff, w_ref[...], (((0,), (1,)), ((), ())),
        preferred_element_type=jnp.float32)
    lj_ref[...] = jnp.sum(jnp.log(dvt_ref[...]), axis=0)[None, None, :]


def _spline_sc_body(d0_hbm, par_hbm, sx_hbm, sy_hbm, dd_hbm, y_hbm, dv_hbm,
                    sx_v, sy_v, dd_v, par_v, x_v, yo_v, dv_v):
    wid = lax.axis_index("s") * 2 + lax.axis_index("c")
    dbase = wid * DPW
    pltpu.sync_copy(par_hbm, par_v)
    pltpu.sync_copy(sx_hbm.at[wid], sx_v.at[pl.ds(0, NCLASS * DPW * NBIN)])
    pltpu.sync_copy(sy_hbm.at[wid], sy_v)
    pltpu.sync_copy(dd_hbm.at[wid], dd_v)
    ngrp = (TCHUNK // 16) * DPW

    def chunk_body(ci, carry):
        t0 = ci * TCHUNK
        pltpu.sync_copy(d0_hbm.at[pl.ds(dbase, DPW), pl.ds(t0, TCHUNK)], x_v)

        @plsc.parallel_loop(0, ngrp, unroll=4)
        def grp_body(g):
            tv = lax.rem(g, TCHUNK // 16)
            dl = lax.div(g, TCHUNK // 16)
            toff = tv * 16
            c_vec = par_v[pl.ds(t0 + toff, 16)]
            base = c_vec * (DPW * NBIN) + dl * NBIN
            v = x_v[dl, pl.ds(toff, 16)]
            idxp = jnp.zeros((16,), jnp.int32)
            basem1 = base - 1
            for b in (64, 32, 16, 8, 4, 2, 1):
                jj = idxp + b
                xprobe = plsc.load_gather(sx_v, [basem1 + jj])
                take = xprobe < v
                if b <= 16:
                    take &= jj <= NBIN
                idxp = jnp.where(take, jj, idxp)
            kk = jnp.clip(idxp - 1, 0, NBIN - 2)
            bk = base + kk
            bk1 = bk + 1
            xk = plsc.load_gather(sx_v, [bk])
            xk1 = plsc.load_gather(sx_v, [bk1])
            yk = plsc.load_gather(sy_v, [bk])
            yk1 = plsc.load_gather(sy_v, [bk1])
            dk = plsc.load_gather(dd_v, [bk])
            dk1 = plsc.load_gather(dd_v, [bk1])
            w = xk1 - xk
            s = (yk1 - yk) / w
            xi = jnp.clip((v - xk) / w, 0.0, 1.0)
            omxi = 1.0 - xi
            xio = xi * omxi
            denom = s + (dk1 + dk - 2.0 * s) * xio
            y_sp = yk + (yk1 - yk) * (s * xi * xi + dk * xio) / denom
            deriv_sp = (s * s
                        * (dk1 * xi * xi + 2.0 * s * xio + dk * omxi * omxi)
                        / (denom * denom))
            below = (idxp == 0) & (v < xk)
            above = idxp >= NBIN
            y_out = jnp.where(below, yk + (v - xk) * dk,
                              jnp.where(above, yk1 + (v - xk1) * dk1, y_sp))
            d_out = jnp.where(below, dk, jnp.where(above, dk1, deriv_sp))
            yo_v[dl, pl.ds(toff, 16)] = y_out
            dv_v[dl, pl.ds(toff, 16)] = d_out

        pltpu.sync_copy(yo_v, y_hbm.at[pl.ds(dbase, DPW), pl.ds(t0, TCHUNK)])
        pltpu.sync_copy(dv_v, dv_hbm.at[pl.ds(dbase, DPW), pl.ds(t0, TCHUNK)])
        return carry

    lax.fori_loop(0, NCHUNK, chunk_body, 0)


def kernel(data, param, wT, knots_x, knots_y, log_deriv):
    param32 = param.astype(jnp.int32)
    kx4 = knots_x.reshape(NCLASS, NWORKER, DPW, NBIN)
    ky4 = knots_y.reshape(NCLASS, NWORKER, DPW, NBIN)
    ld4 = log_deriv.reshape(NCLASS, NWORKER, DPW, NBIN)

    iblk = pl.BlockSpec((NCLASS, 1, DPW, NBIN), lambda i: (0, i, 0, 0))
    oblk = pl.BlockSpec((1, NCLASS, DPW, NBIN), lambda i: (i, 0, 0, 0))
    sx, sy, dd = pl.pallas_call(
        _sort_exp_body,
        grid=(NWORKER,),
        in_specs=[iblk, iblk, iblk],
        out_specs=[oblk, oblk, oblk],
        out_shape=[jax.ShapeDtypeStruct((NWORKER, NCLASS, DPW, NBIN),
                                        jnp.float32)] * 3,
    )(kx4, ky4, ld4)
    sx = sx.reshape(NWORKER, NCLASS * DPW * NBIN)
    sy = sy.reshape(NWORKER, NCLASS * DPW * NBIN)
    dd = dd.reshape(NWORKER, NCLASS * DPW * NBIN)

    data0t = pl.pallas_call(
        _mm_body,
        grid=(NTOK // _MM_T,),
        in_specs=[pl.BlockSpec((NDIM, NDIM), lambda i: (0, 0)),
                  pl.BlockSpec((_MM_T, NDIM), lambda i: (i, 0))],
        out_specs=pl.BlockSpec((NDIM, _MM_T), lambda i: (0, i)),
        out_shape=jax.ShapeDtypeStruct((NDIM, NTOK), jnp.float32),
    )(wT, data)

    spline = pl.kernel(
        _spline_sc_body,
        out_type=[jax.ShapeDtypeStruct((NDIM, NTOK), jnp.float32),
                  jax.ShapeDtypeStruct((NDIM, NTOK), jnp.float32)],
        mesh=plsc.VectorSubcoreMesh(core_axis_name="c", subcore_axis_name="s"),
        compiler_params=pltpu.CompilerParams(use_tc_tiling_on_sc=False,
                                             needs_layout_passes=False),
        scratch_types=[
            pltpu.VMEM((NCLASS * DPW * NBIN + 32,), jnp.float32),
            pltpu.VMEM((NCLASS * DPW * NBIN,), jnp.float32),
            pltpu.VMEM((NCLASS * DPW * NBIN,), jnp.float32),
            pltpu.VMEM((NTOK,), jnp.int32),
            pltpu.VMEM((DPW, TCHUNK), jnp.float32),
            pltpu.VMEM((DPW, TCHUNK), jnp.float32),
            pltpu.VMEM((DPW, TCHUNK), jnp.float32),
        ],
    )
    yt, dvt = spline(data0t, param32, sx, sy, dd)

    data_out, lj = pl.pallas_call(
        _final_body,
        grid=(NTOK // _MM_T,),
        in_specs=[pl.BlockSpec((_MM_T, NDIM), lambda i: (i, 0)),
                  pl.BlockSpec((NDIM, _MM_T), lambda i: (0, i)),
                  pl.BlockSpec((NDIM, _MM_T), lambda i: (0, i)),
                  pl.BlockSpec((NDIM, _MM_T), lambda i: (0, i)),
                  pl.BlockSpec((NDIM, NDIM), lambda i: (0, 0))],
        out_specs=[pl.BlockSpec((_MM_T, NDIM), lambda i: (i, 0)),
                   pl.BlockSpec((1, 1, _MM_T), lambda i: (i, 0, 0))],
        out_shape=[jax.ShapeDtypeStruct((NTOK, NDIM), jnp.float32),
                   jax.ShapeDtypeStruct((NTOK // _MM_T, 1, _MM_T), jnp.float32)],
    )(data, yt, data0t, dvt, wT)
    return data_out, lj.reshape(NTOK)


# speedup vs baseline: 1.2487x; 1.1147x over previous
import functools

import jax
import jax.numpy as jnp
from jax import lax
from jax.experimental import pallas as pl
from jax.experimental.pallas import tpu as pltpu
from jax.experimental.pallas import tpu_sc as plsc

NDIM = 768
NCLASS = 16
NBIN = 100
NTOK = 4096
NBIN_PAD = 128
NROWS = NCLASS * NDIM

NWORKER = 32
DPW = NDIM // NWORKER
TCHUNK = 64
NCHUNK = NTOK // TCHUNK

_SORT_ROWS = 512
_MM_T = 512


_SROWS = NCLASS * DPW


def _sort_exp_body(kx_ref, ky_ref, ld_ref, sx_ref, sy_ref, dd_ref):
    ids = lax.broadcasted_iota(jnp.int32, (_SROWS, NBIN_PAD), 1)
    fill = jnp.full((_SROWS, NBIN_PAD - NBIN), jnp.inf, jnp.float32)

    def bitonic(x):
        x = jnp.concatenate([x.reshape(_SROWS, NBIN), fill], axis=1)
        for k in (2, 4, 8, 16, 32, 64, 128):
            j = k // 2
            while j >= 1:
                pj = pltpu.roll(x, NBIN_PAD - j, axis=1)
                mj = pltpu.roll(x, j, axis=1)
                low = (ids & j) == 0
                partner = jnp.where(low, pj, mj)
                keep_min = low == ((ids & k) == 0)
                x = jnp.where(keep_min, jnp.minimum(x, partner),
                              jnp.maximum(x, partner))
                j //= 2
        return x[:, :NBIN].reshape(1, NCLASS, DPW, NBIN)

    sx_ref[...] = bitonic(kx_ref[...])
    sy_ref[...] = bitonic(ky_ref[...])
    dd_ref[...] = jnp.exp(ld_ref[...]).reshape(1, NCLASS, DPW, NBIN)


def _mm_body(w_ref, a_ref, o_ref):
    o_ref[...] = lax.dot_general(
        w_ref[...], a_ref[...], (((0,), (1,)), ((), ())),
        preferred_element_type=jnp.float32)


def _final_body(data_ref, yt_ref, d0t_ref, dvt_ref, w_ref, o_ref, lj_ref):
    diff = yt_ref[...] - d0t_ref[...]
    o_ref[...] = data_ref[...] + lax.dot_general(
        diff, w_ref[...], (((0,), (1,)), ((), ())),
        preferred_element_type=jnp.float32)
    lj_ref[...] = jnp.sum(jnp.log(dvt_ref[...]), axis=0)[None, None, :]


def _spline_sc_body(d0_hbm, par_hbm, sx_hbm, sy_hbm, dd_hbm, y_hbm, dv_hbm,
                    sx_v, sy_v, dd_v, par_v, x_a, x_b, yo_a, yo_b, dv_a, dv_b,
                    in_a, in_b, out_a, out_b):
    wid = lax.axis_index("s") * 2 + lax.axis_index("c")
    dbase = wid * DPW
    pltpu.sync_copy(par_hbm, par_v)
    pltpu.sync_copy(sx_hbm.at[wid], sx_v.at[pl.ds(0, NCLASS * DPW * NBIN)])
    pltpu.sync_copy(sy_hbm.at[wid], sy_v)
    pltpu.sync_copy(dd_hbm.at[wid], dd_v)
    ngrp = (TCHUNK // 16) * DPW

    def start_in(t0, x_v, sem):
        pltpu.async_copy(d0_hbm.at[pl.ds(dbase, DPW), pl.ds(t0, TCHUNK)],
                         x_v, sem)

    def wait_in(x_v, sem):
        pltpu.make_async_copy(d0_hbm.at[pl.ds(dbase, DPW), pl.ds(0, TCHUNK)],
                              x_v, sem).wait()

    def start_out(t0, yo_v, dv_v, sem):
        pltpu.async_copy(yo_v, y_hbm.at[pl.ds(dbase, DPW), pl.ds(t0, TCHUNK)],
                         sem)
        pltpu.async_copy(dv_v, dv_hbm.at[pl.ds(dbase, DPW), pl.ds(t0, TCHUNK)],
                         sem)

    def wait_out(yo_v, dv_v, sem):
        dst = y_hbm.at[pl.ds(dbase, DPW), pl.ds(0, TCHUNK)]
        pltpu.make_async_copy(yo_v, dst, sem).wait()
        pltpu.make_async_copy(dv_v, dst, sem).wait()

    def compute_chunk(t0, x_v, yo_v, dv_v):
        @plsc.parallel_loop(0, ngrp, unroll=4)
        def grp_body(g):
            tv = lax.div(g, DPW)
            dl = lax.rem(g, DPW)
            toff = tv * 16
            c_vec = par_v[pl.ds(t0 + toff, 16)]
            base = c_vec * (DPW * NBIN) + dl * NBIN
            v = x_v[dl, pl.ds(toff, 16)]
            idxp = jnp.zeros((16,), jnp.int32)
            basem1 = base - 1
            for b in (64, 32, 16, 8, 4, 2, 1):
                jj = idxp + b
                xprobe = plsc.load_gather(sx_v, [basem1 + jj])
                take = xprobe < v
                if b <= 16:
                    take &= jj <= NBIN
                idxp = jnp.where(take, jj, idxp)
            kk = jnp.clip(idxp - 1, 0, NBIN - 2)
            bk = base + kk
            bk1 = bk + 1
            xk = plsc.load_gather(sx_v, [bk])
            xk1 = plsc.load_gather(sx_v, [bk1])
            yk = plsc.load_gather(sy_v, [bk])
            yk1 = plsc.load_gather(sy_v, [bk1])
            dk = plsc.load_gather(dd_v, [bk])
            dk1 = plsc.load_gather(dd_v, [bk1])
            w = xk1 - xk
            s = (yk1 - yk) / w
            xi = jnp.clip((v - xk) / w, 0.0, 1.0)
            omxi = 1.0 - xi
            xio = xi * omxi
            denom = s + (dk1 + dk - 2.0 * s) * xio
            y_sp = yk + (yk1 - yk) * (s * xi * xi + dk * xio) / denom
            deriv_sp = (s * s
                        * (dk1 * xi * xi + 2.0 * s * xio + dk * omxi * omxi)
                        / (denom * denom))
            below = (idxp == 0) & (v < xk)
            above = idxp >= NBIN
            y_out = jnp.where(below, yk + (v - xk) * dk,
                              jnp.where(above, yk1 + (v - xk1) * dk1, y_sp))
            d_out = jnp.where(below, dk, jnp.where(above, dk1, deriv_sp))
            yo_v[dl, pl.ds(toff, 16)] = y_out
            dv_v[dl, pl.ds(toff, 16)] = d_out

    npair = NCHUNK // 2
    start_in(0, x_a, in_a)

    def pair_body(p, carry):
        t0a = (2 * p) * TCHUNK
        t0b = t0a + TCHUNK
        wait_in(x_a, in_a)
        start_in(t0b, x_b, in_b)

        @pl.when(p > 0)
        def _():
            wait_out(yo_a, dv_a, out_a)

        compute_chunk(t0a, x_a, yo_a, dv_a)
        start_out(t0a, yo_a, dv_a, out_a)

        wait_in(x_b, in_b)

        @pl.when(p < npair - 1)
        def _():
            start_in(t0a + 2 * TCHUNK, x_a, in_a)

        @pl.when(p > 0)
        def _():
            wait_out(yo_b, dv_b, out_b)

        compute_chunk(t0b, x_b, yo_b, dv_b)
        start_out(t0b, yo_b, dv_b, out_b)
        return carry

    lax.fori_loop(0, npair, pair_body, 0)
    wait_out(yo_a, dv_a, out_a)
    wait_out(yo_b, dv_b, out_b)


def kernel(data, param, wT, knots_x, knots_y, log_deriv):
    param32 = param.astype(jnp.int32)
    kx4 = knots_x.reshape(NCLASS, NWORKER, DPW, NBIN)
    ky4 = knots_y.reshape(NCLASS, NWORKER, DPW, NBIN)
    ld4 = log_deriv.reshape(NCLASS, NWORKER, DPW, NBIN)

    iblk = pl.BlockSpec((NCLASS, 1, DPW, NBIN), lambda i: (0, i, 0, 0))
    oblk = pl.BlockSpec((1, NCLASS, DPW, NBIN), lambda i: (i, 0, 0, 0))
    sx, sy, dd = pl.pallas_call(
        _sort_exp_body,
        grid=(NWORKER,),
        in_specs=[iblk, iblk, iblk],
        out_specs=[oblk, oblk, oblk],
        out_shape=[jax.ShapeDtypeStruct((NWORKER, NCLASS, DPW, NBIN),
                                        jnp.float32)] * 3,
    )(kx4, ky4, ld4)
    sx = sx.reshape(NWORKER, NCLASS * DPW * NBIN)
    sy = sy.reshape(NWORKER, NCLASS * DPW * NBIN)
    dd = dd.reshape(NWORKER, NCLASS * DPW * NBIN)

    data0t = pl.pallas_call(
        _mm_body,
        grid=(NTOK // _MM_T,),
        in_specs=[pl.BlockSpec((NDIM, NDIM), lambda i: (0, 0)),
                  pl.BlockSpec((_MM_T, NDIM), lambda i: (i, 0))],
        out_specs=pl.BlockSpec((NDIM, _MM_T), lambda i: (0, i)),
        out_shape=jax.ShapeDtypeStruct((NDIM, NTOK), jnp.float32),
    )(wT, data)

    spline = pl.kernel(
        _spline_sc_body,
        out_type=[jax.ShapeDtypeStruct((NDIM, NTOK), jnp.float32),
                  jax.ShapeDtypeStruct((NDIM, NTOK), jnp.float32)],
        mesh=plsc.VectorSubcoreMesh(core_axis_name="c", subcore_axis_name="s"),
        compiler_params=pltpu.CompilerParams(use_tc_tiling_on_sc=False,
                                             needs_layout_passes=False),
        scratch_types=[
            pltpu.VMEM((NCLASS * DPW * NBIN + 32,), jnp.float32),
            pltpu.VMEM((NCLASS * DPW * NBIN,), jnp.float32),
            pltpu.VMEM((NCLASS * DPW * NBIN,), jnp.float32),
            pltpu.VMEM((NTOK,), jnp.int32),
            pltpu.VMEM((DPW, TCHUNK), jnp.float32),
            pltpu.VMEM((DPW, TCHUNK), jnp.float32),
            pltpu.VMEM((DPW, TCHUNK), jnp.float32),
            pltpu.VMEM((DPW, TCHUNK), jnp.float32),
            pltpu.VMEM((DPW, TCHUNK), jnp.float32),
            pltpu.VMEM((DPW, TCHUNK), jnp.float32),
            pltpu.SemaphoreType.DMA,
            pltpu.SemaphoreType.DMA,
            pltpu.SemaphoreType.DMA,
            pltpu.SemaphoreType.DMA,
        ],
    )
    yt, dvt = spline(data0t, param32, sx, sy, dd)

    data_out, lj = pl.pallas_call(
        _final_body,
        grid=(NTOK // _MM_T,),
        in_specs=[pl.BlockSpec((_MM_T, NDIM), lambda i: (i, 0)),
                  pl.BlockSpec((NDIM, _MM_T), lambda i: (0, i)),
                  pl.BlockSpec((NDIM, _MM_T), lambda i: (0, i)),
                  pl.BlockSpec((NDIM, _MM_T), lambda i: (0, i)),
                  pl.BlockSpec((NDIM, NDIM), lambda i: (0, 0))],
        out_specs=[pl.BlockSpec((_MM_T, NDIM), lambda i: (i, 0)),
                   pl.BlockSpec((1, 1, _MM_T), lambda i: (i, 0, 0))],
        out_shape=[jax.ShapeDtypeStruct((NTOK, NDIM), jnp.float32),
                   jax.ShapeDtypeStruct((NTOK // _MM_T, 1, _MM_T), jnp.float32)],
    )(data, yt, data0t, dvt, wT)
    return data_out, lj.reshape(NTOK)


# speedup vs baseline: 1.3143x; 1.0525x over previous
import functools

import jax
import jax.numpy as jnp
from jax import lax
from jax.experimental import pallas as pl
from jax.experimental.pallas import tpu as pltpu
from jax.experimental.pallas import tpu_sc as plsc

NDIM = 768
NCLASS = 16
NBIN = 100
NTOK = 4096
NBIN_PAD = 128
NROWS = NCLASS * NDIM

NWORKER = 32
DPW = NDIM // NWORKER
TCHUNK = 64
NCHUNK = NTOK // TCHUNK

_SORT_ROWS = 512
_MM_T = 512


_SROWS = NCLASS * DPW


def _sort_exp_body(kx_ref, ky_ref, ld_ref, sx_ref, sy_ref, dd_ref):
    ids = lax.broadcasted_iota(jnp.int32, (2 * _SROWS, NBIN_PAD), 1)
    fill = jnp.full((2 * _SROWS, NBIN_PAD - NBIN), jnp.inf, jnp.float32)

    x = jnp.concatenate([kx_ref[...].reshape(_SROWS, NBIN),
                         ky_ref[...].reshape(_SROWS, NBIN)], axis=0)
    x = jnp.concatenate([x, fill], axis=1)
    for k in (2, 4, 8, 16, 32, 64, 128):
        j = k // 2
        while j >= 1:
            pj = pltpu.roll(x, NBIN_PAD - j, axis=1)
            mj = pltpu.roll(x, j, axis=1)
            low = (ids & j) == 0
            partner = jnp.where(low, pj, mj)
            keep_min = low == ((ids & k) == 0)
            x = jnp.where(keep_min, jnp.minimum(x, partner),
                          jnp.maximum(x, partner))
            j //= 2
    res = x[:, :NBIN]
    sx_ref[...] = res[:_SROWS].reshape(1, NCLASS, DPW, NBIN)
    sy_ref[...] = res[_SROWS:].reshape(1, NCLASS, DPW, NBIN)
    dd_ref[...] = jnp.exp(ld_ref[...]).reshape(1, NCLASS, DPW, NBIN)


def _mm_body(w_ref, a_ref, o_ref):
    o_ref[...] = lax.dot_general(
        w_ref[...], a_ref[...], (((0,), (1,)), ((), ())),
        preferred_element_type=jnp.float32)


def _final_body(data_ref, yt_ref, d0t_ref, dvt_ref, w_ref, o_ref, lj_ref):
    diff = yt_ref[...] - d0t_ref[...]
    o_ref[...] = data_ref[...] + lax.dot_general(
        diff, w_ref[...], (((0,), (1,)), ((), ())),
        preferred_element_type=jnp.float32)
    lj_ref[...] = jnp.sum(jnp.log(dvt_ref[...]), axis=0)[None, None, :]


def _spline_sc_body(d0_hbm, par_hbm, sx_hbm, sy_hbm, dd_hbm, y_hbm, dv_hbm,
                    sx_v, sy_v, dd_v, par_v, x_a, x_b, yo_a, yo_b, dv_a, dv_b,
                    in_a, in_b, out_a, out_b):
    wid = lax.axis_index("s") * 2 + lax.axis_index("c")
    dbase = wid * DPW
    pltpu.sync_copy(par_hbm, par_v)
    pltpu.sync_copy(sx_hbm.at[wid], sx_v.at[pl.ds(0, NCLASS * DPW * NBIN)])
    pltpu.sync_copy(sy_hbm.at[wid], sy_v)
    pltpu.sync_copy(dd_hbm.at[wid], dd_v)
    ngrp = (TCHUNK // 16) * DPW

    def start_in(t0, x_v, sem):
        pltpu.async_copy(d0_hbm.at[pl.ds(dbase, DPW), pl.ds(t0, TCHUNK)],
                         x_v, sem)

    def wait_in(x_v, sem):
        pltpu.make_async_copy(d0_hbm.at[pl.ds(dbase, DPW), pl.ds(0, TCHUNK)],
                              x_v, sem).wait()

    def start_out(t0, yo_v, dv_v, sem):
        pltpu.async_copy(yo_v, y_hbm.at[pl.ds(dbase, DPW), pl.ds(t0, TCHUNK)],
                         sem)
        pltpu.async_copy(dv_v, dv_hbm.at[pl.ds(dbase, DPW), pl.ds(t0, TCHUNK)],
                         sem)

    def wait_out(yo_v, dv_v, sem):
        dst = y_hbm.at[pl.ds(dbase, DPW), pl.ds(0, TCHUNK)]
        pltpu.make_async_copy(yo_v, dst, sem).wait()
        pltpu.make_async_copy(dv_v, dst, sem).wait()

    def compute_chunk(t0, x_v, yo_v, dv_v):
        @plsc.parallel_loop(0, ngrp, unroll=4)
        def grp_body(g):
            tv = lax.div(g, DPW)
            dl = lax.rem(g, DPW)
            toff = tv * 16
            c_vec = par_v[pl.ds(t0 + toff, 16)]
            base = c_vec * (DPW * NBIN) + dl * NBIN
            v = x_v[dl, pl.ds(toff, 16)]
            idxp = jnp.zeros((16,), jnp.int32)
            basem1 = base - 1
            for b in (64, 32, 16, 8, 4, 2, 1):
                jj = idxp + b
                xprobe = plsc.load_gather(sx_v, [basem1 + jj])
                take = xprobe < v
                if b <= 16:
                    take &= jj <= NBIN
                idxp = jnp.where(take, jj, idxp)
            kk = jnp.clip(idxp - 1, 0, NBIN - 2)
            bk = base + kk
            bk1 = bk + 1
            xk = plsc.load_gather(sx_v, [bk])
            xk1 = plsc.load_gather(sx_v, [bk1])
            yk = plsc.load_gather(sy_v, [bk])
            yk1 = plsc.load_gather(sy_v, [bk1])
            dk = plsc.load_gather(dd_v, [bk])
            dk1 = plsc.load_gather(dd_v, [bk1])
            w = xk1 - xk
            s = (yk1 - yk) / w
            xi = jnp.clip((v - xk) / w, 0.0, 1.0)
            omxi = 1.0 - xi
            xio = xi * omxi
            denom = s + (dk1 + dk - 2.0 * s) * xio
            y_sp = yk + (yk1 - yk) * (s * xi * xi + dk * xio) / denom
            deriv_sp = (s * s
                        * (dk1 * xi * xi + 2.0 * s * xio + dk * omxi * omxi)
                        / (denom * denom))
            below = (idxp == 0) & (v < xk)
            above = idxp >= NBIN
            y_out = jnp.where(below, yk + (v - xk) * dk,
                              jnp.where(above, yk1 + (v - xk1) * dk1, y_sp))
            d_out = jnp.where(below, dk, jnp.where(above, dk1, deriv_sp))
            yo_v[dl, pl.ds(toff, 16)] = y_out
            dv_v[dl, pl.ds(toff, 16)] = d_out

    npair = NCHUNK // 2
    start_in(0, x_a, in_a)

    def pair_body(p, carry):
        t0a = (2 * p) * TCHUNK
        t0b = t0a + TCHUNK
        wait_in(x_a, in_a)
        start_in(t0b, x_b, in_b)

        @pl.when(p > 0)
        def _():
            wait_out(yo_a, dv_a, out_a)

        compute_chunk(t0a, x_a, yo_a, dv_a)
        start_out(t0a, yo_a, dv_a, out_a)

        wait_in(x_b, in_b)

        @pl.when(p < npair - 1)
        def _():
            start_in(t0a + 2 * TCHUNK, x_a, in_a)

        @pl.when(p > 0)
        def _():
            wait_out(yo_b, dv_b, out_b)

        compute_chunk(t0b, x_b, yo_b, dv_b)
        start_out(t0b, yo_b, dv_b, out_b)
        return carry

    lax.fori_loop(0, npair, pair_body, 0)
    wait_out(yo_a, dv_a, out_a)
    wait_out(yo_b, dv_b, out_b)


def kernel(data, param, wT, knots_x, knots_y, log_deriv):
    param32 = param.astype(jnp.int32)
    kx4 = knots_x.reshape(NCLASS, NWORKER, DPW, NBIN)
    ky4 = knots_y.reshape(NCLASS, NWORKER, DPW, NBIN)
    ld4 = log_deriv.reshape(NCLASS, NWORKER, DPW, NBIN)

    iblk = pl.BlockSpec((NCLASS, 1, DPW, NBIN), lambda i: (0, i, 0, 0))
    oblk = pl.BlockSpec((1, NCLASS, DPW, NBIN), lambda i: (i, 0, 0, 0))
    sx, sy, dd = pl.pallas_call(
        _sort_exp_body,
        grid=(NWORKER,),
        in_specs=[iblk, iblk, iblk],
        out_specs=[oblk, oblk, oblk],
        out_shape=[jax.ShapeDtypeStruct((NWORKER, NCLASS, DPW, NBIN),
                                        jnp.float32)] * 3,
    )(kx4, ky4, ld4)
    sx = sx.reshape(NWORKER, NCLASS * DPW * NBIN)
    sy = sy.reshape(NWORKER, NCLASS * DPW * NBIN)
    dd = dd.reshape(NWORKER, NCLASS * DPW * NBIN)

    data0t = pl.pallas_call(
        _mm_body,
        grid=(NTOK // _MM_T,),
        in_specs=[pl.BlockSpec((NDIM, NDIM), lambda i: (0, 0)),
                  pl.BlockSpec((_MM_T, NDIM), lambda i: (i, 0))],
        out_specs=pl.BlockSpec((NDIM, _MM_T), lambda i: (0, i)),
        out_shape=jax.ShapeDtypeStruct((NDIM, NTOK), jnp.float32),
    )(wT, data)

    spline = pl.kernel(
        _spline_sc_body,
        out_type=[jax.ShapeDtypeStruct((NDIM, NTOK), jnp.float32),
                  jax.ShapeDtypeStruct((NDIM, NTOK), jnp.float32)],
        mesh=plsc.VectorSubcoreMesh(core_axis_name="c", subcore_axis_name="s"),
        compiler_params=pltpu.CompilerParams(use_tc_tiling_on_sc=False,
                                             needs_layout_passes=False),
        scratch_types=[
            pltpu.VMEM((NCLASS * DPW * NBIN + 32,), jnp.float32),
            pltpu.VMEM((NCLASS * DPW * NBIN,), jnp.float32),
            pltpu.VMEM((NCLASS * DPW * NBIN,), jnp.float32),
            pltpu.VMEM((NTOK,), jnp.int32),
            pltpu.VMEM((DPW, TCHUNK), jnp.float32),
            pltpu.VMEM((DPW, TCHUNK), jnp.float32),
            pltpu.VMEM((DPW, TCHUNK), jnp.float32),
            pltpu.VMEM((DPW, TCHUNK), jnp.float32),
            pltpu.VMEM((DPW, TCHUNK), jnp.float32),
            pltpu.VMEM((DPW, TCHUNK), jnp.float32),
            pltpu.SemaphoreType.DMA,
            pltpu.SemaphoreType.DMA,
            pltpu.SemaphoreType.DMA,
            pltpu.SemaphoreType.DMA,
        ],
    )
    yt, dvt = spline(data0t, param32, sx, sy, dd)

    data_out, lj = pl.pallas_call(
        _final_body,
        grid=(NTOK // _MM_T,),
        in_specs=[pl.BlockSpec((_MM_T, NDIM), lambda i: (i, 0)),
                  pl.BlockSpec((NDIM, _MM_T), lambda i: (0, i)),
                  pl.BlockSpec((NDIM, _MM_T), lambda i: (0, i)),
                  pl.BlockSpec((NDIM, _MM_T), lambda i: (0, i)),
                  pl.BlockSpec((NDIM, NDIM), lambda i: (0, 0))],
        out_specs=[pl.BlockSpec((_MM_T, NDIM), lambda i: (i, 0)),
                   pl.BlockSpec((1, 1, _MM_T), lambda i: (i, 0, 0))],
        out_shape=[jax.ShapeDtypeStruct((NTOK, NDIM), jnp.float32),
                   jax.ShapeDtypeStruct((NTOK // _MM_T, 1, _MM_T), jnp.float32)],
    )(data, yt, data0t, dvt, wT)
    return data_out, lj.reshape(NTOK)
